# Initial kernel scaffold; baseline (speedup 1.0000x reference)
import jax
import jax.numpy as jnp
from jax.experimental import pallas as pl


def kernel(x, edge_index, W_self1, W_neigh1, b1, gamma1, beta1, W_self2, W_neigh2, b2):
    raise NotImplementedError("write your pallas kernel here")


# speedup vs baseline: 7.9247x; 7.9247x over previous
import functools

import jax
import jax.numpy as jnp
from jax import lax
from jax.experimental import pallas as pl
from jax.experimental.pallas import tpu as pltpu
from jax.experimental.pallas import tpu_sc as plsc

_N = 10000
_E = 320000
_D = 128
_H = 128
_O = 64

_NC = 2
_NS = 16
_NW = _NC * _NS
_EW = _E // _NW
_C = 80
_K = _EW // _C
_NP = 10240
_RP = _NP // _NS


def _make_sc_pass(width, with_deg):
  mesh = plsc.VectorSubcoreMesh(core_axis_name="c", subcore_axis_name="s")
  out_type = [jax.ShapeDtypeStruct((_NC, _NP, width), jnp.float32)]
  scratch = [
      pltpu.VMEM((_K, _C), jnp.int32),
      pltpu.VMEM((_K, _C), jnp.int32),
      pltpu.VMEM((_C, width), jnp.float32),
      pltpu.VMEM_SHARED((_NP, width), jnp.float32),
      pltpu.SemaphoreType.DMA,
  ]
  if with_deg:
    out_type.append(jax.ShapeDtypeStruct((_NC, _NP), jnp.float32))
    scratch += [
        pltpu.VMEM((_C,), jnp.float32),
        pltpu.VMEM((_RP,), jnp.float32),
        pltpu.VMEM_SHARED((_NP,), jnp.float32),
    ]

  def body(table, src_hbm, dst_hbm, *refs):
    if with_deg:
      (out_agg, out_deg, src_v, dst_v, rows_v, acc_sh, sem,
       ones_v, zdeg_v, deg_sh) = refs
    else:
      out_agg, src_v, dst_v, rows_v, acc_sh, sem = refs
    c = lax.axis_index("c")
    s = lax.axis_index("s")
    wid = c * _NS + s
    base = s * _RP

    npack = width // 16

    def zrow(t, carry):
      rows_v[t // npack, pl.ds((t % npack) * 16, 16)] = jnp.zeros(
          (16,), jnp.float32)
      return carry

    lax.fori_loop(0, _C * npack, zrow, 0)
    for k in range(_RP // _C):
      pltpu.sync_copy(rows_v, acc_sh.at[pl.ds(base + k * _C, _C)])

    if with_deg:
      def zdeg(t, carry):
        zdeg_v[pl.ds(t * 16, 16)] = jnp.zeros((16,), jnp.float32)
        return carry

      lax.fori_loop(0, _RP // 16, zdeg, 0)
      pltpu.sync_copy(zdeg_v, deg_sh.at[pl.ds(base, _RP)])

      def ones(t, carry):
        ones_v[pl.ds(t * 16, 16)] = jnp.ones((16,), jnp.float32)
        return carry

      lax.fori_loop(0, _C // 16, ones, 0)

    plsc.subcore_barrier()

    pltpu.sync_copy(src_hbm.at[wid], src_v)
    pltpu.sync_copy(dst_hbm.at[wid], dst_v)

    def chunk(j, carry):
      pltpu.async_copy(table.at[src_v.at[j]], rows_v, sem).wait()
      pltpu.sync_copy(rows_v, acc_sh.at[dst_v.at[j]], add=True)
      if with_deg:
        pltpu.sync_copy(ones_v, deg_sh.at[dst_v.at[j]], add=True)
      return carry

    lax.fori_loop(0, _K, chunk, 0)

    plsc.subcore_barrier()

    for k in range(_RP // 128):
      sl = pl.ds(base + k * 128, 128)
      pltpu.sync_copy(acc_sh.at[sl], out_agg.at[c, sl])
    if with_deg:
      pltpu.sync_copy(deg_sh.at[pl.ds(base, _RP)],
                      out_deg.at[c, pl.ds(base, _RP)])

  return pl.kernel(body, out_type=tuple(out_type), mesh=mesh,
                   scratch_types=scratch,
                   compiler_params=pltpu.CompilerParams(
                       use_tc_tiling_on_sc=False))


_sc_pass1 = _make_sc_pass(_D, True)
_sc_pass2 = _make_sc_pass(_O, False)

_BR = 1024
_GRID = _NP // _BR


def _tc_a_body(x_ref, ws1, wn1, sb, cb, agg, degt, wn2, h_ref, hw2_ref):
  d = jnp.maximum(degt[:, 0:1] + degt[:, 1:2], 1.0)
  hn = (agg[0] + agg[1]) / d
  hl = jnp.dot(x_ref[...], ws1[...], preferred_element_type=jnp.float32)
  hl = hl + jnp.dot(hn, wn1[...], preferred_element_type=jnp.float32)
  h = jnp.maximum(hl * sb[...] + cb[...], 0.0)
  h_ref[...] = h
  hw2_ref[...] = jnp.dot(h, wn2[...], preferred_element_type=jnp.float32)


_tc_a = pl.pallas_call(
    _tc_a_body,
    grid=(_GRID,),
    in_specs=[
        pl.BlockSpec((_BR, _D), lambda i: (i, 0)),
        pl.BlockSpec((_D, _H), lambda i: (0, 0)),
        pl.BlockSpec((_D, _H), lambda i: (0, 0)),
        pl.BlockSpec((1, _H), lambda i: (0, 0)),
        pl.BlockSpec((1, _H), lambda i: (0, 0)),
        pl.BlockSpec((_NC, _BR, _D), lambda i: (0, i, 0)),
        pl.BlockSpec((_BR, _NC), lambda i: (i, 0)),
        pl.BlockSpec((_H, _O), lambda i: (0, 0)),
    ],
    out_specs=[
        pl.BlockSpec((_BR, _H), lambda i: (i, 0)),
        pl.BlockSpec((_BR, _O), lambda i: (i, 0)),
    ],
    out_shape=[
        jax.ShapeDtypeStruct((_N, _H), jnp.float32),
        jax.ShapeDtypeStruct((_N, _O), jnp.float32),
    ],
)


def _tc_b_body(h_ref, ws2, agg2, degt, b2, out_ref):
  d = jnp.maximum(degt[:, 0:1] + degt[:, 1:2], 1.0)
  hn2 = (agg2[0] + agg2[1]) / d
  out_ref[...] = (
      jnp.dot(h_ref[...], ws2[...], preferred_element_type=jnp.float32)
      + hn2 + b2[...])


_tc_b = pl.pallas_call(
    _tc_b_body,
    grid=(_GRID,),
    in_specs=[
        pl.BlockSpec((_BR, _H), lambda i: (i, 0)),
        pl.BlockSpec((_H, _O), lambda i: (0, 0)),
        pl.BlockSpec((_NC, _BR, _O), lambda i: (0, i, 0)),
        pl.BlockSpec((_BR, _NC), lambda i: (i, 0)),
        pl.BlockSpec((1, _O), lambda i: (0, 0)),
    ],
    out_specs=pl.BlockSpec((_BR, _O), lambda i: (i, 0)),
    out_shape=jax.ShapeDtypeStruct((_N, _O), jnp.float32),
)


def kernel(x, edge_index, W_self1, W_neigh1, b1, gamma1, beta1,
           W_self2, W_neigh2, b2):
  src = edge_index[0].reshape(_NW, _K, _C)
  dst = edge_index[1].reshape(_NW, _K, _C)

  aggx, deg = _sc_pass1(x, src, dst)
  degt = deg.T

  sb = (gamma1 * (1.0 / jnp.sqrt(1.0 + 1e-5))).reshape(1, _H)
  cb = (b1 * sb[0] + beta1).reshape(1, _H)

  h, hw2 = _tc_a(x, W_self1, W_neigh1, sb, cb, aggx, degt, W_neigh2)
  (agg2,) = _sc_pass2(hw2, src, dst)
  out = _tc_b(h, W_self2, agg2, degt, b2.reshape(1, _O))
  return out
